# baseline (speedup 1.0000x reference)
import functools

import jax
import jax.numpy as jnp
from jax import lax
from jax.experimental import pallas as pl
from jax.experimental.pallas import tpu as pltpu
from jax.experimental.pallas import tpu_sc as plsc

NC, NS, L = 2, 16, 16
NW = NC * NS
B, S, D = 1024, 200, 128
N = B * S
TPW = N // NW
C = 64
NCHUNK = TPW // C
NB = 4
NJ = D // L
GR = 8
EPS = 1e-5

_mesh = plsc.VectorSubcoreMesh(core_axis_name="c", subcore_axis_name="s")


def _rsqrt(v):
    y = lax.bitcast_convert_type(
        jnp.full((L,), 0x5F3759DF, jnp.int32)
        - (lax.bitcast_convert_type(v, jnp.int32) >> 1),
        jnp.float32,
    )
    for _ in range(2):
        y = y * (1.5 - 0.5 * v * y * y)
    return y


_DN = lax.GatherDimensionNumbers(
    offset_dims=(), collapsed_slice_dims=(0,), start_index_map=(0,))


def _gather16(vec, idx):
    return lax.gather(vec, idx[:, None], _DN, slice_sizes=(1,),
                      mode=lax.GatherScatterMode.PROMISE_IN_BOUNDS)


def _allsum(v, perms):
    for pm in perms:
        v = v + _gather16(v, pm)
    return v


def _body(x_ref, seg_ref, tok_ref, pos2_ref, sege_ref, gam_ref, bet_ref, out_ref,
          idx_v, seg_v, bufs0, bufs1, bufs2, bufs3, par_v,
          psem0, psem1, psem2, psem3, gsem0, gsem1, gsem2, gsem3,
          osem0, osem1, osem2, osem3):
    bufs = (bufs0, bufs1, bufs2, bufs3)
    psems = (psem0, psem1, psem2, psem3)
    gsems = (gsem0, gsem1, gsem2, gsem3)
    osems = (osem0, osem1, osem2, osem3)
    wid = lax.axis_index("s") * NC + lax.axis_index("c")
    base_tok = wid * TPW

    pltpu.sync_copy(x_ref.at[pl.ds(base_tok, TPW)], idx_v)
    pltpu.sync_copy(seg_ref.at[pl.ds(base_tok, TPW)], seg_v)
    pltpu.sync_copy(gam_ref, par_v.at[0])
    pltpu.sync_copy(bet_ref, par_v.at[1])

    gam = [par_v[0, pl.ds(j * L, L)] for j in range(NJ)]
    bet = [par_v[1, pl.ds(j * L, L)] for j in range(NJ)]
    perms = [jnp.arange(L, dtype=jnp.int32) ^ k for k in (8, 4, 2, 1)]

    def _prefill(c, k):
        pbase = (c * C) % S
        pltpu.async_copy(pos2_ref.at[pl.ds(pbase, C)], bufs[k], psems[k])

    def _wait_pre(k):
        pltpu.make_async_copy(
            pos2_ref.at[pl.ds(0, C)], bufs[k], psems[k]).wait()

    def _gadd(c, k):
        pltpu.async_copy(tok_ref.at[idx_v.at[pl.ds(c * C, C)]], bufs[k],
                         gsems[k], add=True)
        pltpu.async_copy(sege_ref.at[seg_v.at[pl.ds(c * C, C)]], bufs[k],
                         gsems[k], add=True)

    def _wait_gadd(k):
        pltpu.make_async_copy(
            tok_ref.at[idx_v.at[pl.ds(0, C)]], bufs[k], gsems[k]).wait()
        pltpu.make_async_copy(
            sege_ref.at[seg_v.at[pl.ds(0, C)]], bufs[k], gsems[k]).wait()

    def _wait_out(k):
        pltpu.make_async_copy(
            bufs[k], out_ref.at[pl.ds(base_tok, C)], osems[k]).wait()

    def _proc(c, k):
        buf = bufs[k]
        _wait_gadd(k)

        def _group(gi, carry2):
            r0 = gi * GR
            for i in range(GR):
                r = r0 + i
                v = [buf[r, pl.ds(j * L, L)] for j in range(NJ)]
                ssum = v[0]
                s2 = v[0] * v[0]
                for j in range(1, NJ):
                    ssum = ssum + v[j]
                    s2 = s2 + v[j] * v[j]
                tot = _allsum(ssum, perms)
                tot2 = _allsum(s2, perms)
                mean = tot * (1.0 / D)
                var = tot2 * (1.0 / D) - mean * mean
                rstd = _rsqrt(var + EPS)
                for j in range(NJ):
                    buf[r, pl.ds(j * L, L)] = ((v[j] - mean) * (rstd * gam[j])
                                               + bet[j])
            return carry2

        lax.fori_loop(0, C // GR, _group, 0)
        pltpu.async_copy(buf, out_ref.at[pl.ds(base_tok + c * C, C)], osems[k])

    _prefill(0, 0)
    _prefill(1, 1)
    _wait_pre(0)
    _gadd(0, 0)

    def _step(c, u):
        ku1 = (u + 1) % NB
        ku2 = (u + 2) % NB

        @pl.when(c >= 2)
        def _wo():
            _wait_out(ku2)

        @pl.when(c + 2 < NCHUNK)
        def _pf():
            _prefill(c + 2, ku2)

        @pl.when(c + 1 < NCHUNK)
        def _ga():
            _wait_pre(ku1)
            _gadd(c + 1, ku1)

        _proc(c, u)

    def _iter4(i, carry):
        for u in range(NB):
            _step(NB * i + u, u)
        return carry

    lax.fori_loop(0, NCHUNK // NB, _iter4, 0)
    _wait_out((NCHUNK - 2) % NB)
    _wait_out((NCHUNK - 1) % NB)


_emb = functools.partial(
    pl.kernel,
    out_type=jax.ShapeDtypeStruct((N, D), jnp.float32),
    mesh=_mesh,
    scratch_types=[
        pltpu.VMEM((TPW,), jnp.int32),
        pltpu.VMEM((TPW,), jnp.int32),
        pltpu.VMEM((C, D), jnp.float32),
        pltpu.VMEM((C, D), jnp.float32),
        pltpu.VMEM((C, D), jnp.float32),
        pltpu.VMEM((C, D), jnp.float32),
        pltpu.VMEM((2, D), jnp.float32),
        pltpu.SemaphoreType.DMA,
        pltpu.SemaphoreType.DMA,
        pltpu.SemaphoreType.DMA,
        pltpu.SemaphoreType.DMA,
        pltpu.SemaphoreType.DMA,
        pltpu.SemaphoreType.DMA,
        pltpu.SemaphoreType.DMA,
        pltpu.SemaphoreType.DMA,
        pltpu.SemaphoreType.DMA,
        pltpu.SemaphoreType.DMA,
        pltpu.SemaphoreType.DMA,
        pltpu.SemaphoreType.DMA,
    ],
)(_body)


def kernel(x, seg, tok_embed, pos_embed, seg_embed, gamma, beta):
    x1 = x.reshape(N).astype(jnp.int32)
    seg1 = seg.reshape(N).astype(jnp.int32)
    pos2 = jnp.concatenate([pos_embed[:S], pos_embed[:S]], axis=0)
    out = _emb(x1, seg1, tok_embed, pos2, seg_embed, gamma, beta)
    return out.reshape(B, S, D)


# speedup vs baseline: 17.6002x; 17.6002x over previous
import functools

import jax
import jax.numpy as jnp
from jax import lax
from jax.experimental import pallas as pl
from jax.experimental.pallas import tpu as pltpu
from jax.experimental.pallas import tpu_sc as plsc

NC, NS, L = 2, 16, 16
NW = NC * NS
B, S, D = 1024, 200, 128
N = B * S
TPW = N // NW
C = 128
NCHUNK = TPW // C
NB = 4
EPS = 1e-5
BB = 8

_mesh = plsc.VectorSubcoreMesh(core_axis_name="c", subcore_axis_name="s")


def _sc_body(x_ref, tok_ref, out_ref, idx_v, bufs0, bufs1, bufs2, bufs3,
             gsem0, gsem1, gsem2, gsem3, osem0, osem1, osem2, osem3):
    bufs = (bufs0, bufs1, bufs2, bufs3)
    gsems = (gsem0, gsem1, gsem2, gsem3)
    osems = (osem0, osem1, osem2, osem3)
    wid = lax.axis_index("s") * NC + lax.axis_index("c")
    base_tok = wid * TPW

    pltpu.sync_copy(x_ref.at[pl.ds(base_tok, TPW)], idx_v)

    def _gather(c, k):
        pltpu.async_copy(tok_ref.at[idx_v.at[pl.ds(c * C, C)]], bufs[k],
                         gsems[k])

    def _wait_gather(k):
        pltpu.make_async_copy(
            tok_ref.at[idx_v.at[pl.ds(0, C)]], bufs[k], gsems[k]).wait()

    def _wait_out(k):
        pltpu.make_async_copy(
            bufs[k], out_ref.at[pl.ds(base_tok, C)], osems[k]).wait()

    def _proc(c, k):
        _wait_gather(k)
        pltpu.async_copy(bufs[k], out_ref.at[pl.ds(base_tok + c * C, C)],
                         osems[k])

    _gather(0, 0)
    _gather(1, 1)

    def _step(c, u):
        ku2 = (u + 2) % NB

        @pl.when(c + 2 < NCHUNK)
        def _ga():
            @pl.when(c >= 2)
            def _wo():
                _wait_out(ku2)
            _gather(c + 2, ku2)

        _proc(c, u)

    def _iter4(i, carry):
        for u in range(NB):
            _step(NB * i + u, u)
        return carry

    lax.fori_loop(0, NCHUNK // NB, _iter4, 0)
    _step(NCHUNK - 2, (NCHUNK - 2) % NB)
    _step(NCHUNK - 1, (NCHUNK - 1) % NB)
    for k in range(NB):
        _wait_out((NCHUNK - NB + 1 + k) % NB)


_sc_gather = functools.partial(
    pl.kernel,
    out_type=jax.ShapeDtypeStruct((N, D), jnp.float32),
    mesh=_mesh,
    scratch_types=[
        pltpu.VMEM((TPW,), jnp.int32),
        pltpu.VMEM((C, D), jnp.float32),
        pltpu.VMEM((C, D), jnp.float32),
        pltpu.VMEM((C, D), jnp.float32),
        pltpu.VMEM((C, D), jnp.float32),
        pltpu.SemaphoreType.DMA,
        pltpu.SemaphoreType.DMA,
        pltpu.SemaphoreType.DMA,
        pltpu.SemaphoreType.DMA,
        pltpu.SemaphoreType.DMA,
        pltpu.SemaphoreType.DMA,
        pltpu.SemaphoreType.DMA,
        pltpu.SemaphoreType.DMA,
    ],
)(_sc_body)


def _ln_body(tok_ref, seg_ref, pos_ref, sege_ref, gam_ref, bet_ref, o_ref):
    t = tok_ref[...]
    g = seg_ref[...]
    pos = pos_ref[...]
    se = sege_ref[...]
    h = (t + pos[None, :, :] + se[0][None, None, :]
         + g[:, :, None] * (se[1] - se[0])[None, None, :])
    mean = jnp.mean(h, axis=-1, keepdims=True)
    cen = h - mean
    var = jnp.mean(cen * cen, axis=-1, keepdims=True)
    o_ref[...] = (cen * lax.rsqrt(var + EPS) * gam_ref[...][None, None, :]
                  + bet_ref[...][None, None, :])


_ln_tc = functools.partial(
    pl.pallas_call,
    out_shape=jax.ShapeDtypeStruct((B, S, D), jnp.float32),
    grid=(B // BB,),
    in_specs=[
        pl.BlockSpec((BB, S, D), lambda i: (i, 0, 0)),
        pl.BlockSpec((BB, S), lambda i: (i, 0)),
        pl.BlockSpec((S, D), lambda i: (0, 0)),
        pl.BlockSpec((2, D), lambda i: (0, 0)),
        pl.BlockSpec((D,), lambda i: (0,)),
        pl.BlockSpec((D,), lambda i: (0,)),
    ],
    out_specs=pl.BlockSpec((BB, S, D), lambda i: (i, 0, 0)),
)(_ln_body)


def kernel(x, seg, tok_embed, pos_embed, seg_embed, gamma, beta):
    x1 = x.reshape(N).astype(jnp.int32)
    rows = _sc_gather(x1, tok_embed)
    segf = seg.astype(jnp.float32)
    out = _ln_tc(rows.reshape(B, S, D), segf, pos_embed[:S], seg_embed,
                 gamma, beta)
    return out


# speedup vs baseline: 20.5379x; 1.1669x over previous
import functools

import jax
import jax.numpy as jnp
from jax import lax
from jax.experimental import pallas as pl
from jax.experimental.pallas import tpu as pltpu
from jax.experimental.pallas import tpu_sc as plsc

NC, NS, L = 2, 16, 16
NW = NC * NS
B, S, D = 1024, 200, 128
N = B * S
TPW = N // NW
C = 128
NCHUNK = TPW // C
NB = 4
EPS = 1e-5
BB = 16

_mesh = plsc.VectorSubcoreMesh(core_axis_name="c", subcore_axis_name="s")


def _sc_body(x_ref, tok_ref, out_ref, idx_v, bufs0, bufs1, bufs2, bufs3,
             gsem0, gsem1, gsem2, gsem3, osem0, osem1, osem2, osem3):
    bufs = (bufs0, bufs1, bufs2, bufs3)
    gsems = (gsem0, gsem1, gsem2, gsem3)
    osems = (osem0, osem1, osem2, osem3)
    wid = lax.axis_index("s") * NC + lax.axis_index("c")
    base_tok = wid * TPW

    pltpu.sync_copy(x_ref.at[pl.ds(base_tok, TPW)], idx_v)

    def _gather(c, k):
        pltpu.async_copy(tok_ref.at[idx_v.at[pl.ds(c * C, C)]], bufs[k],
                         gsems[k])

    def _wait_gather(k):
        pltpu.make_async_copy(
            tok_ref.at[idx_v.at[pl.ds(0, C)]], bufs[k], gsems[k]).wait()

    def _wait_out(k):
        pltpu.make_async_copy(
            bufs[k], out_ref.at[pl.ds(base_tok, C)], osems[k]).wait()

    def _proc(c, k):
        _wait_gather(k)
        pltpu.async_copy(bufs[k], out_ref.at[pl.ds(base_tok + c * C, C)],
                         osems[k])

    _gather(0, 0)
    _gather(1, 1)

    def _step(c, u):
        ku2 = (u + 2) % NB

        @pl.when(c + 2 < NCHUNK)
        def _ga():
            @pl.when(c >= 2)
            def _wo():
                _wait_out(ku2)
            _gather(c + 2, ku2)

        _proc(c, u)

    def _iter4(i, carry):
        for u in range(NB):
            _step(NB * i + u, u)
        return carry

    lax.fori_loop(0, NCHUNK // NB, _iter4, 0)
    _step(NCHUNK - 2, (NCHUNK - 2) % NB)
    _step(NCHUNK - 1, (NCHUNK - 1) % NB)
    for k in range(NB):
        _wait_out((NCHUNK - NB + 1 + k) % NB)


_sc_gather = functools.partial(
    pl.kernel,
    out_type=jax.ShapeDtypeStruct((N, D), jnp.float32),
    mesh=_mesh,
    scratch_types=[
        pltpu.VMEM((TPW,), jnp.int32),
        pltpu.VMEM((C, D), jnp.float32),
        pltpu.VMEM((C, D), jnp.float32),
        pltpu.VMEM((C, D), jnp.float32),
        pltpu.VMEM((C, D), jnp.float32),
        pltpu.SemaphoreType.DMA,
        pltpu.SemaphoreType.DMA,
        pltpu.SemaphoreType.DMA,
        pltpu.SemaphoreType.DMA,
        pltpu.SemaphoreType.DMA,
        pltpu.SemaphoreType.DMA,
        pltpu.SemaphoreType.DMA,
        pltpu.SemaphoreType.DMA,
    ],
)(_sc_body)


def _ln_body(tok_ref, seg_ref, pos_ref, sege_ref, gam_ref, bet_ref, o_ref):
    t = tok_ref[...]
    g = seg_ref[...]
    pos = pos_ref[...]
    se = sege_ref[...]
    h = (t + pos[None, :, :] + se[0][None, None, :]
         + g[:, :, None] * (se[1] - se[0])[None, None, :])
    mean = jnp.mean(h, axis=-1, keepdims=True)
    cen = h - mean
    var = jnp.mean(cen * cen, axis=-1, keepdims=True)
    o_ref[...] = (cen * lax.rsqrt(var + EPS) * gam_ref[...][None, None, :]
                  + bet_ref[...][None, None, :])


_ln_tc = functools.partial(
    pl.pallas_call,
    out_shape=jax.ShapeDtypeStruct((B, S, D), jnp.float32),
    grid=(B // BB,),
    in_specs=[
        pl.BlockSpec((BB, S, D), lambda i: (i, 0, 0)),
        pl.BlockSpec((BB, S), lambda i: (i, 0)),
        pl.BlockSpec((S, D), lambda i: (0, 0)),
        pl.BlockSpec((2, D), lambda i: (0, 0)),
        pl.BlockSpec((D,), lambda i: (0,)),
        pl.BlockSpec((D,), lambda i: (0,)),
    ],
    out_specs=pl.BlockSpec((BB, S, D), lambda i: (i, 0, 0)),
)(_ln_body)


def kernel(x, seg, tok_embed, pos_embed, seg_embed, gamma, beta):
    x1 = x.reshape(N).astype(jnp.int32)
    rows = _sc_gather(x1, tok_embed)
    segf = seg.astype(jnp.float32)
    out = _ln_tc(rows.reshape(B, S, D), segf, pos_embed[:S], seg_embed,
                 gamma, beta)
    return out


# speedup vs baseline: 22.2913x; 1.0854x over previous
import functools

import jax
import jax.numpy as jnp
from jax import lax
from jax.experimental import pallas as pl
from jax.experimental.pallas import tpu as pltpu
from jax.experimental.pallas import tpu_sc as plsc

NC, NS, L = 2, 16, 16
NW = NC * NS
B, S, D = 1024, 200, 128
N = B * S
TPW = N // NW
C = 128
NCHUNK = TPW // C
NB = 4
EPS = 1e-5
BB = 32

_mesh = plsc.VectorSubcoreMesh(core_axis_name="c", subcore_axis_name="s")


def _sc_body(x_ref, tok_ref, out_ref, idx_v, bufs0, bufs1, bufs2, bufs3,
             gsem0, gsem1, gsem2, gsem3, osem0, osem1, osem2, osem3):
    bufs = (bufs0, bufs1, bufs2, bufs3)
    gsems = (gsem0, gsem1, gsem2, gsem3)
    osems = (osem0, osem1, osem2, osem3)
    wid = lax.axis_index("s") * NC + lax.axis_index("c")
    base_tok = wid * TPW

    pltpu.sync_copy(x_ref.at[pl.ds(base_tok, TPW)], idx_v)

    def _gather(c, k):
        pltpu.async_copy(tok_ref.at[idx_v.at[pl.ds(c * C, C)]], bufs[k],
                         gsems[k])

    def _wait_gather(k):
        pltpu.make_async_copy(
            tok_ref.at[idx_v.at[pl.ds(0, C)]], bufs[k], gsems[k]).wait()

    def _wait_out(k):
        pltpu.make_async_copy(
            bufs[k], out_ref.at[pl.ds(base_tok, C)], osems[k]).wait()

    def _proc(c, k):
        _wait_gather(k)
        pltpu.async_copy(bufs[k], out_ref.at[pl.ds(base_tok + c * C, C)],
                         osems[k])

    _gather(0, 0)
    _gather(1, 1)

    def _step(c, u):
        ku2 = (u + 2) % NB

        @pl.when(c + 2 < NCHUNK)
        def _ga():
            @pl.when(c >= 2)
            def _wo():
                _wait_out(ku2)
            _gather(c + 2, ku2)

        _proc(c, u)

    def _iter4(i, carry):
        for u in range(NB):
            _step(NB * i + u, u)
        return carry

    lax.fori_loop(0, NCHUNK // NB, _iter4, 0)
    _step(NCHUNK - 2, (NCHUNK - 2) % NB)
    _step(NCHUNK - 1, (NCHUNK - 1) % NB)
    for k in range(NB):
        _wait_out((NCHUNK - NB + 1 + k) % NB)


_sc_gather = functools.partial(
    pl.kernel,
    out_type=jax.ShapeDtypeStruct((N, D), jnp.float32),
    mesh=_mesh,
    scratch_types=[
        pltpu.VMEM((TPW,), jnp.int32),
        pltpu.VMEM((C, D), jnp.float32),
        pltpu.VMEM((C, D), jnp.float32),
        pltpu.VMEM((C, D), jnp.float32),
        pltpu.VMEM((C, D), jnp.float32),
        pltpu.SemaphoreType.DMA,
        pltpu.SemaphoreType.DMA,
        pltpu.SemaphoreType.DMA,
        pltpu.SemaphoreType.DMA,
        pltpu.SemaphoreType.DMA,
        pltpu.SemaphoreType.DMA,
        pltpu.SemaphoreType.DMA,
        pltpu.SemaphoreType.DMA,
    ],
)(_sc_body)


def _ln_body(tok_ref, seg_ref, pos_ref, sege_ref, gam_ref, bet_ref, o_ref):
    t = tok_ref[...]
    g = seg_ref[...]
    pos = pos_ref[...]
    se = sege_ref[...]
    h = (t + pos[None, :, :] + se[0][None, None, :]
         + g[:, :, None] * (se[1] - se[0])[None, None, :])
    mean = jnp.mean(h, axis=-1, keepdims=True)
    cen = h - mean
    var = jnp.mean(cen * cen, axis=-1, keepdims=True)
    o_ref[...] = (cen * lax.rsqrt(var + EPS) * gam_ref[...][None, None, :]
                  + bet_ref[...][None, None, :])


_ln_tc = functools.partial(
    pl.pallas_call,
    out_shape=jax.ShapeDtypeStruct((B, S, D), jnp.float32),
    grid=(B // BB,),
    in_specs=[
        pl.BlockSpec((BB, S, D), lambda i: (i, 0, 0)),
        pl.BlockSpec((BB, S), lambda i: (i, 0)),
        pl.BlockSpec((S, D), lambda i: (0, 0)),
        pl.BlockSpec((2, D), lambda i: (0, 0)),
        pl.BlockSpec((D,), lambda i: (0,)),
        pl.BlockSpec((D,), lambda i: (0,)),
    ],
    out_specs=pl.BlockSpec((BB, S, D), lambda i: (i, 0, 0)),
)(_ln_body)


def kernel(x, seg, tok_embed, pos_embed, seg_embed, gamma, beta):
    x1 = x.reshape(N).astype(jnp.int32)
    rows = _sc_gather(x1, tok_embed)
    segf = seg.astype(jnp.float32)
    out = _ln_tc(rows.reshape(B, S, D), segf, pos_embed[:S], seg_embed,
                 gamma, beta)
    return out
